# baseline (speedup 1.0000x reference)
import functools

import jax
import jax.numpy as jnp
from jax import lax
from jax.experimental import pallas as pl
from jax.experimental.pallas import tpu as pltpu
from jax.experimental.pallas import tpu_sc as plsc

_B = 4096
_D = 128
_V = 100000


def _sc_gather(target_word, emb_table):
    info = plsc.get_sparse_core_info()
    nc, ns = info.num_cores, info.num_subcores
    nw = nc * ns
    b_per_w = _B // nw
    mesh = plsc.VectorSubcoreMesh(core_axis_name="c", subcore_axis_name="s")

    @functools.partial(
        pl.kernel,
        mesh=mesh,
        out_type=jax.ShapeDtypeStruct((_B, _D), jnp.float32),
        scratch_types=[
            pltpu.VMEM((b_per_w,), jnp.int32),
            pltpu.VMEM((b_per_w, _D), jnp.float32),
            pltpu.SemaphoreType.DMA,
        ],
    )
    def gather_kernel(idx_hbm, table_hbm, out_hbm, idx_v, rows_v, sem):
        wid = lax.axis_index("s") * nc + lax.axis_index("c")
        base = wid * b_per_w
        pltpu.sync_copy(idx_hbm.at[pl.ds(base, b_per_w)], idx_v)
        pltpu.async_copy(table_hbm.at[idx_v], rows_v, sem).wait()
        pltpu.sync_copy(rows_v, out_hbm.at[pl.ds(base, b_per_w)])

    return gather_kernel(target_word, emb_table)


_BM = 32
_NS = _B // _BM
_STR = _BM // 8
_WMAIN = 99968
_WTAIL = _V - _WMAIN
_NC = 4096
_CHUNKS = [_NC] * (_WMAIN // _NC) + [_WMAIN - (_WMAIN // _NC) * _NC]


def _mm_kernel(emb_ref, w_ref, b_ref, out_hbm, acc, colbuf, sems, tsem):
    i = pl.program_id(0)
    slot = lax.rem(i, 2)

    @pl.when(i >= 2)
    def _wait_prev():
        for k in range(_STR):
            pltpu.make_async_copy(
                acc.at[slot, pl.ds(8 * k, 8), :],
                out_hbm.at[pl.ds(8 * k, 8), pl.ds(0, _WMAIN)],
                sems.at[slot, k],
            ).wait()

    emb_blk = emb_ref[pl.ds(i * _BM, _BM), :]

    off = 0
    for width in _CHUNKS:
        acc[slot, :, pl.ds(off, width)] = lax.dot_general(
            emb_blk, w_ref[pl.ds(off, width), :],
            dimension_numbers=(((1,), (1,)), ((), ())),
            preferred_element_type=jnp.float32,
        ) + b_ref[:, pl.ds(off, width)]
        off += width

    colbuf[pl.ds(i * _BM, _BM), :] = lax.dot_general(
        emb_blk, w_ref[pl.ds(_WMAIN, _WTAIL), :],
        dimension_numbers=(((1,), (1,)), ((), ())),
        preferred_element_type=jnp.float32,
    ) + b_ref[:, pl.ds(_WMAIN, _WTAIL)]

    for k in range(_STR):
        pltpu.make_async_copy(
            acc.at[slot, pl.ds(8 * k, 8), :],
            out_hbm.at[pl.ds(i * _BM + 8 * k, 8), pl.ds(0, _WMAIN)],
            sems.at[slot, k],
        ).start()

    @pl.when(i == _NS - 1)
    def _drain():
        pltpu.make_async_copy(
            colbuf, out_hbm.at[:, pl.ds(_WMAIN, _WTAIL)], tsem,
        ).start()
        for s in range(2):
            for k in range(_STR):
                pltpu.make_async_copy(
                    acc.at[s, pl.ds(8 * k, 8), :],
                    out_hbm.at[pl.ds(8 * k, 8), pl.ds(0, _WMAIN)],
                    sems.at[s, k],
                ).wait()
        pltpu.make_async_copy(
            colbuf, out_hbm.at[:, pl.ds(_WMAIN, _WTAIL)], tsem,
        ).wait()


def _tc_project(embedded_bf16, W_bf16, b2d):
    return pl.pallas_call(
        _mm_kernel,
        grid=(_NS,),
        in_specs=[
            pl.BlockSpec((_B, _D), lambda i: (0, 0)),
            pl.BlockSpec((_V, _D), lambda i: (0, 0)),
            pl.BlockSpec((1, _V), lambda i: (0, 0)),
        ],
        out_specs=pl.BlockSpec(memory_space=pl.ANY),
        out_shape=jax.ShapeDtypeStruct((_B, _V), jnp.float32),
        scratch_shapes=[
            pltpu.VMEM((2, _BM, _WMAIN), jnp.float32),
            pltpu.VMEM((_B, _WTAIL), jnp.float32),
            pltpu.SemaphoreType.DMA((2, _STR)),
            pltpu.SemaphoreType.DMA,
        ],
    )(embedded_bf16, W_bf16, b2d)


def kernel(target_word, emb_table, W, b):
    embedded = _sc_gather(target_word.astype(jnp.int32), emb_table)
    return _tc_project(
        embedded.astype(jnp.bfloat16),
        W.astype(jnp.bfloat16),
        b.reshape(1, _V),
    )


# speedup vs baseline: 1.2380x; 1.2380x over previous
import functools

import jax
import jax.numpy as jnp
from jax import lax
from jax.experimental import pallas as pl
from jax.experimental.pallas import tpu as pltpu
from jax.experimental.pallas import tpu_sc as plsc

_B = 4096
_D = 128
_V = 100000


def _sc_gather(target_word, emb_table):
    info = plsc.get_sparse_core_info()
    nc, ns = info.num_cores, info.num_subcores
    nw = nc * ns
    b_per_w = _B // nw
    mesh = plsc.VectorSubcoreMesh(core_axis_name="c", subcore_axis_name="s")

    @functools.partial(
        pl.kernel,
        mesh=mesh,
        out_type=jax.ShapeDtypeStruct((_B, _D), jnp.float32),
        scratch_types=[
            pltpu.VMEM((b_per_w,), jnp.int32),
            pltpu.VMEM((b_per_w, _D), jnp.float32),
            pltpu.SemaphoreType.DMA,
        ],
    )
    def gather_kernel(idx_hbm, table_hbm, out_hbm, idx_v, rows_v, sem):
        wid = lax.axis_index("s") * nc + lax.axis_index("c")
        base = wid * b_per_w
        pltpu.sync_copy(idx_hbm.at[pl.ds(base, b_per_w)], idx_v)
        pltpu.async_copy(table_hbm.at[idx_v], rows_v, sem).wait()
        pltpu.sync_copy(rows_v, out_hbm.at[pl.ds(base, b_per_w)])

    return gather_kernel(target_word, emb_table)


_BM = 512
_MT = _B // _BM
_BN = 8192
_NTF = 12
_TW = 1664
_WTAIL = _V - _NTF * _BN - _TW
_STR = _BM // 8
_NC = 2048


def _accum_into(emb_blk, w_ref, b_ref, dst_ref, widths):
    off = 0
    for wdt in widths:
        dst_ref[:, pl.ds(off, wdt)] = lax.dot_general(
            emb_blk, w_ref[pl.ds(off, wdt), :],
            dimension_numbers=(((1,), (1,)), ((), ())),
            preferred_element_type=jnp.float32,
        ) + b_ref[:, pl.ds(off, wdt)]
        off += wdt


def _mm_kernel(emb_ref, w_ref, b_ref, out_hbm,
               acc, tailbuf, colbuf, sems, tsem, csem):
    i = pl.program_id(0)
    j = pl.program_id(1)
    f_ord = i * _NTF + j
    slot = lax.rem(f_ord, 2)

    emb_blk = emb_ref[pl.ds(i * _BM, _BM), :]

    @pl.when(jnp.logical_and(j < _NTF, f_ord >= 2))
    def _wait_prev_full():
        for k in range(_STR):
            pltpu.make_async_copy(
                acc.at[slot, pl.ds(8 * k, 8), :],
                out_hbm.at[pl.ds(8 * k, 8), pl.ds(0, _BN)],
                sems.at[slot],
            ).wait()

    @pl.when(j < _NTF)
    def _full_tile():
        _accum_into(emb_blk, w_ref, b_ref, acc.at[slot],
                    [_NC] * (_BN // _NC))
        for k in range(_STR):
            pltpu.make_async_copy(
                acc.at[slot, pl.ds(8 * k, 8), :],
                out_hbm.at[pl.ds(i * _BM + 8 * k, 8), pl.ds(j * _BN, _BN)],
                sems.at[slot],
            ).start()

    @pl.when(j == _NTF)
    def _tail_tile():
        @pl.when(i >= 1)
        def _wait_prev_tail():
            for k in range(_STR):
                pltpu.make_async_copy(
                    tailbuf.at[pl.ds(8 * k, 8), :],
                    out_hbm.at[pl.ds(8 * k, 8), pl.ds(_NTF * _BN, _TW)],
                    tsem,
                ).wait()

        _accum_into(emb_blk, w_ref, b_ref, tailbuf, [1024, _TW - 1024])
        colbuf[pl.ds(i * _BM, _BM), :] = lax.dot_general(
            emb_blk, w_ref[pl.ds(_TW, _WTAIL), :],
            dimension_numbers=(((1,), (1,)), ((), ())),
            preferred_element_type=jnp.float32,
        ) + b_ref[:, pl.ds(_TW, _WTAIL)]
        for k in range(_STR):
            pltpu.make_async_copy(
                tailbuf.at[pl.ds(8 * k, 8), :],
                out_hbm.at[pl.ds(i * _BM + 8 * k, 8), pl.ds(_NTF * _BN, _TW)],
                tsem,
            ).start()

    @pl.when(jnp.logical_and(i == _MT - 1, j == _NTF))
    def _drain():
        pltpu.make_async_copy(
            colbuf, out_hbm.at[:, pl.ds(_NTF * _BN + _TW, _WTAIL)], csem,
        ).start()
        for s in range(2):
            for k in range(_STR):
                pltpu.make_async_copy(
                    acc.at[s, pl.ds(8 * k, 8), :],
                    out_hbm.at[pl.ds(8 * k, 8), pl.ds(0, _BN)],
                    sems.at[s],
                ).wait()
        for k in range(_STR):
            pltpu.make_async_copy(
                tailbuf.at[pl.ds(8 * k, 8), :],
                out_hbm.at[pl.ds(8 * k, 8), pl.ds(_NTF * _BN, _TW)],
                tsem,
            ).wait()
        pltpu.make_async_copy(
            colbuf, out_hbm.at[:, pl.ds(_NTF * _BN + _TW, _WTAIL)], csem,
        ).wait()


def _tc_project(embedded_bf16, W_bf16, b2d):
    return pl.pallas_call(
        _mm_kernel,
        grid=(_MT, _NTF + 1),
        in_specs=[
            pl.BlockSpec((_B, _D), lambda i, j: (0, 0)),
            pl.BlockSpec((_BN, _D), lambda i, j: (j, 0)),
            pl.BlockSpec((1, _BN), lambda i, j: (0, j)),
        ],
        out_specs=pl.BlockSpec(memory_space=pl.ANY),
        out_shape=jax.ShapeDtypeStruct((_B, _V), jnp.float32),
        scratch_shapes=[
            pltpu.VMEM((2, _BM, _BN), jnp.float32),
            pltpu.VMEM((_BM, _TW), jnp.float32),
            pltpu.VMEM((_B, _WTAIL), jnp.float32),
            pltpu.SemaphoreType.DMA((2,)),
            pltpu.SemaphoreType.DMA,
            pltpu.SemaphoreType.DMA,
        ],
    )(embedded_bf16, W_bf16, b2d)


def kernel(target_word, emb_table, W, b):
    embedded = _sc_gather(target_word.astype(jnp.int32), emb_table)
    return _tc_project(
        embedded.astype(jnp.bfloat16),
        W.astype(jnp.bfloat16),
        b.reshape(1, _V),
    )
